# baseline (speedup 1.0000x reference)
import jax
import jax.numpy as jnp
from jax.experimental import pallas as pl
from jax.experimental.pallas import tpu as pltpu

S = 2048
D = 1024


N_CHUNK = 8
CS = S // N_CHUNK


def _body(bos_ref, idx_ref, embeds_ref, speech_ref, pos_ref, out_ref,
          row_a, row_b, row_c, sem_bulk, sem_a, sem_b, sem_c0, sem_c1):
    bulk = []
    for b in range(2):
        for c in range(N_CHUNK):
            cp = pltpu.make_async_copy(
                embeds_ref.at[b, pl.ds(c * CS, CS), :],
                out_ref.at[b, pl.ds(c * CS, CS), :],
                sem_bulk.at[b * N_CHUNK + c])
            cp.start()
            bulk.append(cp)

    tok = bos_ref[0, 0]
    ix = idx_ref[0]
    cp_a = pltpu.make_async_copy(speech_ref.at[pl.ds(tok, 1), :], row_a, sem_a)
    cp_b = pltpu.make_async_copy(pos_ref.at[pl.ds(ix, 1), :], row_b, sem_b)
    cp_a.start()
    cp_b.start()
    cp_a.wait()
    cp_b.wait()
    row_c[...] = row_a[...] + row_b[...]

    cp0 = pltpu.make_async_copy(row_c, out_ref.at[0, pl.ds(S, 1), :], sem_c0)
    cp1 = pltpu.make_async_copy(row_c, out_ref.at[1, pl.ds(S, 1), :], sem_c1)
    cp0.start()
    cp1.start()
    cp0.wait()
    cp1.wait()
    for cp in bulk:
        cp.wait()


def kernel(bos_token, embeds, idx, speech_emb, pos_emb):
    out = pl.pallas_call(
        _body,
        out_shape=jax.ShapeDtypeStruct((2, S + 1, D), jnp.float32),
        in_specs=[
            pl.BlockSpec(memory_space=pltpu.SMEM),
            pl.BlockSpec(memory_space=pltpu.SMEM),
            pl.BlockSpec(memory_space=pl.ANY),
            pl.BlockSpec(memory_space=pl.ANY),
            pl.BlockSpec(memory_space=pl.ANY),
        ],
        out_specs=pl.BlockSpec(memory_space=pl.ANY),
        scratch_shapes=[
            pltpu.VMEM((1, D), jnp.float32),
            pltpu.VMEM((1, D), jnp.float32),
            pltpu.VMEM((1, D), jnp.float32),
            pltpu.SemaphoreType.DMA((2 * N_CHUNK,)),
            pltpu.SemaphoreType.DMA,
            pltpu.SemaphoreType.DMA,
            pltpu.SemaphoreType.DMA,
            pltpu.SemaphoreType.DMA,
        ],
    )(bos_token, idx, embeds, speech_emb, pos_emb)
    return out


# speedup vs baseline: 9.4788x; 9.4788x over previous
import jax
import jax.numpy as jnp
from jax.experimental import pallas as pl
from jax.experimental.pallas import tpu as pltpu

S = 2048
D = 1024
BS = 256
NB = S // BS


def _body(bos_ref, idx_ref, embeds_ref, speech_ref, pos_ref, out_ref,
          row_a, row_b, sem_a, sem_b):
    i = pl.program_id(0)

    @pl.when(i < NB)
    def _copy():
        out_ref[...] = embeds_ref[...]

    @pl.when(i == NB)
    def _bos():
        tok = bos_ref[0, 0]
        ix = idx_ref[0]
        cp_a = pltpu.make_async_copy(speech_ref.at[pl.ds(tok, 1), :], row_a,
                                     sem_a)
        cp_b = pltpu.make_async_copy(pos_ref.at[pl.ds(ix, 1), :], row_b,
                                     sem_b)
        cp_a.start()
        cp_b.start()
        cp_a.wait()
        cp_b.wait()
        row = row_a[...] + row_b[...]
        out_ref[0, pl.ds(0, 1), :] = row
        out_ref[1, pl.ds(0, 1), :] = row


def kernel(bos_token, embeds, idx, speech_emb, pos_emb):
    out = pl.pallas_call(
        _body,
        grid=(NB + 1,),
        out_shape=jax.ShapeDtypeStruct((2, S + 1, D), jnp.float32),
        in_specs=[
            pl.BlockSpec(memory_space=pltpu.SMEM),
            pl.BlockSpec(memory_space=pltpu.SMEM),
            pl.BlockSpec((2, BS, D), lambda i: (0, jnp.minimum(i, NB - 1), 0)),
            pl.BlockSpec(memory_space=pl.ANY),
            pl.BlockSpec(memory_space=pl.ANY),
        ],
        out_specs=pl.BlockSpec((2, BS, D), lambda i: (0, i, 0)),
        scratch_shapes=[
            pltpu.VMEM((1, D), jnp.float32),
            pltpu.VMEM((1, D), jnp.float32),
            pltpu.SemaphoreType.DMA,
            pltpu.SemaphoreType.DMA,
        ],
    )(bos_token, idx, embeds, speech_emb, pos_emb)
    return out


# speedup vs baseline: 9.5077x; 1.0031x over previous
import jax
import jax.numpy as jnp
from jax.experimental import pallas as pl
from jax.experimental.pallas import tpu as pltpu

S = 2048
D = 1024
BS = 256
NB = S // BS


def _body(bos_ref, idx_ref, embeds_ref, speech_ref, pos_ref, out_ref,
          row_a, row_b, sem_a, sem_b):
    i = pl.program_id(0)
    tok = bos_ref[0, 0]
    ix = idx_ref[0]
    cp_a = pltpu.make_async_copy(speech_ref.at[pl.ds(tok, 1), :], row_a, sem_a)
    cp_b = pltpu.make_async_copy(pos_ref.at[pl.ds(ix, 1), :], row_b, sem_b)

    @pl.when(i == 0)
    def _start_gather():
        cp_a.start()
        cp_b.start()

    @pl.when(i < NB)
    def _copy():
        out_ref[...] = embeds_ref[...]

    @pl.when(i == NB)
    def _bos():
        cp_a.wait()
        cp_b.wait()
        row = row_a[...] + row_b[...]
        out_ref[0, pl.ds(0, 1), :] = row
        out_ref[1, pl.ds(0, 1), :] = row


def kernel(bos_token, embeds, idx, speech_emb, pos_emb):
    out = pl.pallas_call(
        _body,
        grid=(NB + 1,),
        out_shape=jax.ShapeDtypeStruct((2, S + 1, D), jnp.float32),
        in_specs=[
            pl.BlockSpec(memory_space=pltpu.SMEM),
            pl.BlockSpec(memory_space=pltpu.SMEM),
            pl.BlockSpec((2, BS, D), lambda i: (0, jnp.minimum(i, NB - 1), 0)),
            pl.BlockSpec(memory_space=pl.ANY),
            pl.BlockSpec(memory_space=pl.ANY),
        ],
        out_specs=pl.BlockSpec((2, BS, D), lambda i: (0, i, 0)),
        scratch_shapes=[
            pltpu.VMEM((1, D), jnp.float32),
            pltpu.VMEM((1, D), jnp.float32),
            pltpu.SemaphoreType.DMA,
            pltpu.SemaphoreType.DMA,
        ],
    )(bos_token, idx, embeds, speech_emb, pos_emb)
    return out


# speedup vs baseline: 9.7030x; 1.0205x over previous
import jax
import jax.numpy as jnp
from jax.experimental import pallas as pl
from jax.experimental.pallas import tpu as pltpu

S = 2048
D = 1024
BS = 512
NB = S // BS


def _body(bos_ref, idx_ref, embeds_ref, speech_ref, pos_ref, out_ref,
          row_a, row_b, sem_a, sem_b):
    i = pl.program_id(0)
    tok = bos_ref[0, 0]
    ix = idx_ref[0]
    cp_a = pltpu.make_async_copy(speech_ref.at[pl.ds(tok, 1), :], row_a, sem_a)
    cp_b = pltpu.make_async_copy(pos_ref.at[pl.ds(ix, 1), :], row_b, sem_b)

    @pl.when(i == 0)
    def _start_gather():
        cp_a.start()
        cp_b.start()

    @pl.when(i < NB)
    def _copy():
        out_ref[...] = embeds_ref[...]

    @pl.when(i == NB)
    def _bos():
        cp_a.wait()
        cp_b.wait()
        row = row_a[...] + row_b[...]
        out_ref[0, pl.ds(0, 1), :] = row
        out_ref[1, pl.ds(0, 1), :] = row


def kernel(bos_token, embeds, idx, speech_emb, pos_emb):
    out = pl.pallas_call(
        _body,
        grid=(NB + 1,),
        out_shape=jax.ShapeDtypeStruct((2, S + 1, D), jnp.float32),
        in_specs=[
            pl.BlockSpec(memory_space=pltpu.SMEM),
            pl.BlockSpec(memory_space=pltpu.SMEM),
            pl.BlockSpec((2, BS, D), lambda i: (0, jnp.minimum(i, NB - 1), 0)),
            pl.BlockSpec(memory_space=pl.ANY),
            pl.BlockSpec(memory_space=pl.ANY),
        ],
        out_specs=pl.BlockSpec((2, BS, D), lambda i: (0, i, 0)),
        scratch_shapes=[
            pltpu.VMEM((1, D), jnp.float32),
            pltpu.VMEM((1, D), jnp.float32),
            pltpu.SemaphoreType.DMA,
            pltpu.SemaphoreType.DMA,
        ],
    )(bos_token, idx, embeds, speech_emb, pos_emb)
    return out


# speedup vs baseline: 9.9337x; 1.0238x over previous
import jax
import jax.numpy as jnp
from jax.experimental import pallas as pl
from jax.experimental.pallas import tpu as pltpu

S = 2048
D = 1024
BS = 1024
NB = S // BS


def _body(bos_ref, idx_ref, embeds_ref, speech_ref, pos_ref, out_ref,
          row_a, row_b, sem_a, sem_b):
    i = pl.program_id(0)
    tok = bos_ref[0, 0]
    ix = idx_ref[0]
    cp_a = pltpu.make_async_copy(speech_ref.at[pl.ds(tok, 1), :], row_a, sem_a)
    cp_b = pltpu.make_async_copy(pos_ref.at[pl.ds(ix, 1), :], row_b, sem_b)

    @pl.when(i == 0)
    def _start_gather():
        cp_a.start()
        cp_b.start()

    @pl.when(i < NB)
    def _copy():
        out_ref[...] = embeds_ref[...]

    @pl.when(i == NB)
    def _bos():
        cp_a.wait()
        cp_b.wait()
        row = row_a[...] + row_b[...]
        out_ref[0, pl.ds(0, 1), :] = row
        out_ref[1, pl.ds(0, 1), :] = row


def kernel(bos_token, embeds, idx, speech_emb, pos_emb):
    out = pl.pallas_call(
        _body,
        grid=(NB + 1,),
        out_shape=jax.ShapeDtypeStruct((2, S + 1, D), jnp.float32),
        in_specs=[
            pl.BlockSpec(memory_space=pltpu.SMEM),
            pl.BlockSpec(memory_space=pltpu.SMEM),
            pl.BlockSpec((2, BS, D), lambda i: (0, jnp.minimum(i, NB - 1), 0)),
            pl.BlockSpec(memory_space=pl.ANY),
            pl.BlockSpec(memory_space=pl.ANY),
        ],
        out_specs=pl.BlockSpec((2, BS, D), lambda i: (0, i, 0)),
        scratch_shapes=[
            pltpu.VMEM((1, D), jnp.float32),
            pltpu.VMEM((1, D), jnp.float32),
            pltpu.SemaphoreType.DMA,
            pltpu.SemaphoreType.DMA,
        ],
    )(bos_token, idx, embeds, speech_emb, pos_emb)
    return out
